# baseline (speedup 1.0000x reference)
import jax
import jax.numpy as jnp
from jax import lax
from jax.experimental import pallas as pl
from jax.experimental.pallas import tpu as pltpu
from jax.experimental.pallas import tpu_sc as plsc

_B = 1024
_W = 200
_D = 64
_V = 1000000
_NC = 2
_NS = 16
_NW = _NC * _NS
_CHUNK = 128
_NCHUNKS = _B * _W // _CHUNK
_CPW = _NCHUNKS // _NW
_NBB = _B // _CHUNK
_WPW = _W // (_NW // _NBB)


def _emb_body(jdx_hbm, par_hbm, table_hbm, out_hbm, jdx_v, par_v, rows_v,
              tb_v, gsem, ssem):
    wid = lax.axis_index("s") * _NC + lax.axis_index("c")
    b_blk = wid // (_NW // _NBB)
    w0 = (wid % (_NW // _NBB)) * _WPW
    pltpu.sync_copy(jdx_hbm.at[wid], jdx_v)
    pltpu.sync_copy(par_hbm.at[wid], par_v)

    iotas = [G * 16 + lax.iota(jnp.int32, 16) for G in range(8)]

    def extract(k, j):
        colbs = [par_v[j, pl.ds(G * 16, 16)] * 64 for G in range(8)]

        @plsc.parallel_loop(0, _D, unroll=8)
        def _d(d):
            d_blk = lax.shift_right_logical(d, 3)
            d_sub = d & 7
            for G in range(8):
                v = plsc.load_gather(rows_v.at[k], [iotas[G], colbs[G] + d])
                tb_v[k, d_blk, d_sub, pl.ds(G * 16, 16)] = v

    def fire_gather(k, j):
        pltpu.async_copy(table_hbm.at[jdx_v.at[j]], rows_v.at[k], gsem.at[k])

    def wait_gather(k):
        pltpu.make_async_copy(
            table_hbm.at[jdx_v.at[0]], rows_v.at[k], gsem.at[k]).wait()

    def fire_store(k, j):
        pltpu.async_copy(tb_v.at[k], out_hbm.at[w0 + j, :, b_blk, :, :],
                         ssem.at[k])

    def wait_store(k):
        pltpu.make_async_copy(tb_v.at[k], out_hbm.at[0, :, 0, :, :],
                              ssem.at[k]).wait()

    fire_gather(0, 0)
    fire_gather(1, 1)

    @pl.loop(0, _CPW // 2)
    def _grp(g):
        for k in range(2):
            j = g * 2 + k
            wait_gather(k)

            @pl.when(g > 0)
            def _():
                wait_store(k)
            extract(k, j)

            @pl.when(g + 1 < _CPW // 2)
            def _():
                fire_gather(k, j + 2)
            fire_store(k, j)

    wait_store(0)
    wait_store(1)


def kernel(token_ids, table):
    ids3 = token_ids.reshape(_NBB, _CHUNK, _W).transpose(0, 2, 1)
    jdx = lax.shift_right_logical(ids3, 1).reshape(_NW, _CPW, _CHUNK)
    par = (ids3 & 1).reshape(_NW, _CPW, _CHUNK)
    wide = table.reshape(_V // 2, 128)
    mesh = plsc.VectorSubcoreMesh(core_axis_name="c", subcore_axis_name="s")
    out5 = pl.kernel(
        _emb_body,
        out_type=jax.ShapeDtypeStruct((_W, 8, _NBB, 8, _CHUNK), jnp.float32),
        mesh=mesh,
        scratch_types=[
            pltpu.VMEM((_CPW, _CHUNK), jnp.int32),
            pltpu.VMEM((_CPW, _CHUNK), jnp.int32),
            pltpu.VMEM((2, _CHUNK, 128), jnp.float32),
            pltpu.VMEM((2, 8, 8, _CHUNK), jnp.float32),
            pltpu.SemaphoreType.DMA((2,)),
            pltpu.SemaphoreType.DMA((2,)),
        ],
        compiler_params=pltpu.CompilerParams(use_tc_tiling_on_sc=True,
                                             needs_layout_passes=False),
    )(jdx, par, wide)
    return out5.transpose(2, 4, 0, 1, 3).reshape(_B, _W, _D)


# speedup vs baseline: 1.0010x; 1.0010x over previous
import jax
import jax.numpy as jnp
from jax import lax
from jax.experimental import pallas as pl
from jax.experimental.pallas import tpu as pltpu
from jax.experimental.pallas import tpu_sc as plsc

_B = 1024
_W = 200
_D = 64
_V = 1000000
_NC = 2
_NS = 16
_NW = _NC * _NS
_CHUNK = 128
_NCHUNKS = _B * _W // _CHUNK
_CPW = _NCHUNKS // _NW
_NBB = _B // _CHUNK
_WPW = _W // (_NW // _NBB)


def _emb_body(jdx_hbm, par_hbm, table_hbm, out_hbm, jdx_v, par_v, rows_v,
              tb_v, gsem, ssem):
    wid = lax.axis_index("s") * _NC + lax.axis_index("c")
    b_blk = wid // (_NW // _NBB)
    w0 = (wid % (_NW // _NBB)) * _WPW
    pltpu.sync_copy(jdx_hbm.at[wid], jdx_v)
    pltpu.sync_copy(par_hbm.at[wid], par_v)

    iotas = [G * 16 + lax.iota(jnp.int32, 16) for G in range(8)]

    def extract(k, j):
        colbs = [par_v[j, pl.ds(G * 16, 16)] * 64 for G in range(8)]

        @plsc.parallel_loop(0, _D, unroll=8)
        def _d(d):
            d_blk = lax.shift_right_logical(d, 3)
            d_sub = d & 7
            for G in range(8):
                v = plsc.load_gather(rows_v.at[k], [iotas[G], colbs[G] + d])
                tb_v[k, d_blk, d_sub, pl.ds(G * 16, 16)] = v

    def fire_gather(k, j):
        pltpu.async_copy(table_hbm.at[jdx_v.at[j]], rows_v.at[k], gsem.at[k])

    def wait_gather(k):
        pltpu.make_async_copy(
            table_hbm.at[jdx_v.at[0]], rows_v.at[k], gsem.at[k]).wait()

    def fire_store(k, j):
        pltpu.async_copy(tb_v.at[k], out_hbm.at[w0 + j, :, b_blk, :, :],
                         ssem.at[k])

    def wait_store(k):
        pltpu.make_async_copy(tb_v.at[k], out_hbm.at[0, :, 0, :, :],
                              ssem.at[k]).wait()

    fire_gather(0, 0)
    fire_gather(1, 1)
    fire_gather(2, 2)

    @pl.loop(0, 16)
    def _grp(g):
        for k in range(3):
            j = g * 3 + k
            wait_gather(k)

            @pl.when(g > 0)
            def _():
                wait_store(k)
            extract(k, j)

            @pl.when(j + 3 < _CPW)
            def _():
                fire_gather(k, j + 3)
            fire_store(k, j)

    for k in range(2):
        j = 48 + k
        wait_gather(k)
        wait_store(k)
        extract(k, j)
        fire_store(k, j)

    wait_store(0)
    wait_store(1)
    wait_store(2)


def kernel(token_ids, table):
    ids3 = token_ids.reshape(_NBB, _CHUNK, _W).transpose(0, 2, 1)
    jdx = lax.shift_right_logical(ids3, 1).reshape(_NW, _CPW, _CHUNK)
    par = (ids3 & 1).reshape(_NW, _CPW, _CHUNK)
    wide = table.reshape(_V // 2, 128)
    mesh = plsc.VectorSubcoreMesh(core_axis_name="c", subcore_axis_name="s")
    out5 = pl.kernel(
        _emb_body,
        out_type=jax.ShapeDtypeStruct((_W, 8, _NBB, 8, _CHUNK), jnp.float32),
        mesh=mesh,
        scratch_types=[
            pltpu.VMEM((_CPW, _CHUNK), jnp.int32),
            pltpu.VMEM((_CPW, _CHUNK), jnp.int32),
            pltpu.VMEM((3, _CHUNK, 128), jnp.float32),
            pltpu.VMEM((3, 8, 8, _CHUNK), jnp.float32),
            pltpu.SemaphoreType.DMA((3,)),
            pltpu.SemaphoreType.DMA((3,)),
        ],
        compiler_params=pltpu.CompilerParams(use_tc_tiling_on_sc=True,
                                             needs_layout_passes=False),
    )(jdx, par, wide)
    return out5.transpose(2, 4, 0, 1, 3).reshape(_B, _W, _D)


# speedup vs baseline: 1.0037x; 1.0027x over previous
import jax
import jax.numpy as jnp
from jax import lax
from jax.experimental import pallas as pl
from jax.experimental.pallas import tpu as pltpu
from jax.experimental.pallas import tpu_sc as plsc

_B = 1024
_W = 200
_D = 64
_N = _B * _W
_NC = 2
_NS = 16
_NW = _NC * _NS
_CHUNK = 128
_NCHUNKS = _N // _CHUNK
_CPW = _NCHUNKS // _NW

_NBUF = 5
_NGRP = _CPW // _NBUF


def _emb_body(idx_hbm, table_hbm, out_hbm, idx_v, rows_v, gsem, ssem):
    wid = lax.axis_index("s") * _NC + lax.axis_index("c")
    pltpu.sync_copy(idx_hbm.at[wid], idx_v)

    @pl.loop(0, _NGRP)
    def _grp(g):
        for k in range(_NBUF):
            @pl.when(g > 0)
            def _():
                pltpu.make_async_copy(
                    rows_v.at[k], out_hbm.at[wid, 0], ssem.at[k]
                ).wait()
            pltpu.async_copy(
                table_hbm.at[idx_v.at[g * _NBUF + k]], rows_v.at[k], gsem.at[k]
            )
        for k in range(_NBUF):
            pltpu.make_async_copy(
                table_hbm.at[idx_v.at[0]], rows_v.at[k], gsem.at[k]
            ).wait()
            pltpu.async_copy(rows_v.at[k], out_hbm.at[wid, g * _NBUF + k],
                             ssem.at[k])

    for k in range(_NBUF):
        pltpu.make_async_copy(
            rows_v.at[k], out_hbm.at[wid, 0], ssem.at[k]
        ).wait()


def kernel(token_ids, table):
    idx = token_ids.reshape(_NW, _CPW, _CHUNK).astype(jnp.int32)
    mesh = plsc.VectorSubcoreMesh(core_axis_name="c", subcore_axis_name="s")
    out = pl.kernel(
        _emb_body,
        out_type=jax.ShapeDtypeStruct((_NW, _CPW, _CHUNK, _D), jnp.float32),
        mesh=mesh,
        scratch_types=[
            pltpu.VMEM((_CPW, _CHUNK), jnp.int32),
            pltpu.VMEM((_NBUF, _CHUNK, _D), jnp.float32),
            pltpu.SemaphoreType.DMA((_NBUF,)),
            pltpu.SemaphoreType.DMA((_NBUF,)),
        ],
        compiler_params=pltpu.CompilerParams(use_tc_tiling_on_sc=False),
    )(idx, table)
    return out.reshape(_B, _W, _D)
